# Initial kernel scaffold; baseline (speedup 1.0000x reference)
import jax
import jax.numpy as jnp
from jax.experimental import pallas as pl


def kernel(x, W, b):
    raise NotImplementedError("write your pallas kernel here")


# speedup vs baseline: 7.3327x; 7.3327x over previous
import functools

import jax
import jax.numpy as jnp
from jax.experimental import pallas as pl

K = 16
ROWS = 256


def _fused_kernel(xr_ref, xf_ref, w1_ref, w2_ref, b_ref, o_ref, *, n, d, k):
    xr = xr_ref[0]
    xf = xf_ref[0]
    w1 = w1_ref[...]
    w2 = w2_ref[...]
    bias = b_ref[...]

    r = xr.shape[0]

    xx_r = jnp.sum(xr * xr, axis=1, keepdims=True)
    xx_f = jnp.sum(xf * xf, axis=1, keepdims=True)
    a = jax.lax.dot_general(
        xr, xf, (((1,), (1,)), ((), ())),
        preferred_element_type=jnp.float32)
    scores = -xx_r - (-2.0 * a) - jnp.reshape(xx_f, (1, n))

    iota = jax.lax.broadcasted_iota(jnp.int32, (r, n), 1)

    def body(_, carry):
        scores, acc = carry
        m = jnp.max(scores, axis=1, keepdims=True)
        eq = scores == m
        idx = jnp.min(jnp.where(eq, iota, n), axis=1, keepdims=True)
        hot = iota == idx
        onehot = hot.astype(jnp.float32)
        scores = jnp.where(hot, -jnp.inf, scores)
        sel = jax.lax.dot_general(
            onehot, xf, (((1,), (0,)), ((), ())),
            preferred_element_type=jnp.float32)
        dif = sel - xr
        v = jax.lax.dot_general(
            dif, w1, (((1,), (1,)), ((), ())),
            preferred_element_type=jnp.float32)
        return scores, jnp.maximum(acc, v)

    acc0 = jnp.full((r, w1.shape[0]), -jnp.inf, dtype=jnp.float32)
    _, acc = jax.lax.fori_loop(0, k, body, (scores, acc0))

    glob = jax.lax.dot_general(
        xr, w2, (((1,), (1,)), ((), ())),
        preferred_element_type=jnp.float32)
    o_ref[0] = acc + glob + bias


def kernel(x, W, b):
    B, d, n = x.shape
    d_out = W.shape[0]
    xt = jnp.transpose(x, (0, 2, 1))
    w1 = W[:, :d]
    w2 = W[:, d:]
    b2 = jnp.reshape(b, (1, d_out))

    grid = (B, n // ROWS)
    out = pl.pallas_call(
        functools.partial(_fused_kernel, n=n, d=d, k=K),
        grid=grid,
        in_specs=[
            pl.BlockSpec((1, ROWS, d), lambda bi, ti: (bi, ti, 0)),
            pl.BlockSpec((1, n, d), lambda bi, ti: (bi, 0, 0)),
            pl.BlockSpec((d_out, d), lambda bi, ti: (0, 0)),
            pl.BlockSpec((d_out, d), lambda bi, ti: (0, 0)),
            pl.BlockSpec((1, d_out), lambda bi, ti: (0, 0)),
        ],
        out_specs=pl.BlockSpec((1, ROWS, d_out), lambda bi, ti: (bi, ti, 0)),
        out_shape=jax.ShapeDtypeStruct((B, n, d_out), jnp.float32),
    )(xt, xt, w1, w2, b2)
    return jnp.transpose(out, (0, 2, 1))
